# Initial kernel scaffold; baseline (speedup 1.0000x reference)
import jax
import jax.numpy as jnp
from jax.experimental import pallas as pl


def kernel(u, grid, edge_index_one, w1, b1, w2, b2, root):
    raise NotImplementedError("write your pallas kernel here")


# speedup vs baseline: 2.4472x; 2.4472x over previous
import functools

import jax
import jax.numpy as jnp
from jax import lax
from jax.experimental import pallas as pl
from jax.experimental.pallas import tpu as pltpu
from jax.experimental.pallas import tpu_sc as plsc

NC = 2
NS = 16
NW = NC * NS
CHUNK = 128


def _gather_call(tbl, src, dst, e_pad, npad, width):
  ew = e_pad // NW
  nch = ew // CHUNK

  mesh = plsc.VectorSubcoreMesh(core_axis_name="c", subcore_axis_name="s")

  @functools.partial(
      pl.kernel,
      out_type=[
          jax.ShapeDtypeStruct((e_pad, width), jnp.float32),
          jax.ShapeDtypeStruct((e_pad, width), jnp.float32),
      ],
      mesh=mesh,
      scratch_types=[
          pltpu.VMEM((CHUNK,), jnp.int32),
          pltpu.VMEM((CHUNK, width), jnp.float32),
          pltpu.SemaphoreType.DMA,
      ],
      compiler_params=pltpu.CompilerParams(use_tc_tiling_on_sc=False),
  )
  def k(tbl_hbm, src_hbm, dst_hbm, a1_hbm, a2_hbm, idx_v, rows_v, sem):
    w = lax.axis_index("s") * NC + lax.axis_index("c")
    base0 = w * ew

    def step(j, carry):
      base = base0 + j * CHUNK
      pltpu.sync_copy(src_hbm.at[pl.ds(base, CHUNK)], idx_v)
      pltpu.async_copy(tbl_hbm.at[idx_v], rows_v, sem).wait()
      pltpu.sync_copy(rows_v, a1_hbm.at[pl.ds(base, CHUNK)])
      pltpu.sync_copy(dst_hbm.at[pl.ds(base, CHUNK)], idx_v)
      pltpu.async_copy(tbl_hbm.at[idx_v], rows_v, sem).wait()
      pltpu.sync_copy(rows_v, a2_hbm.at[pl.ds(base, CHUNK)])
      return carry

    lax.fori_loop(0, nch, step, 0)

  return k(tbl, src, dst)


def _scatter_call(msg, dsti, zrows, ones, e_pad, npad):
  ew = e_pad // NW
  nch = ew // CHUNK
  rpt = npad // NS

  mesh = plsc.VectorSubcoreMesh(core_axis_name="c", subcore_axis_name="s")

  @functools.partial(
      pl.kernel,
      out_type=[
          jax.ShapeDtypeStruct((NC * npad, 16), jnp.float32),
          jax.ShapeDtypeStruct((NC * npad, 16), jnp.float32),
      ],
      mesh=mesh,
      scratch_types=[
          pltpu.VMEM((CHUNK,), jnp.int32),
          pltpu.VMEM((CHUNK, 16), jnp.float32),
          pltpu.VMEM((CHUNK, 16), jnp.float32),
          pltpu.VMEM_SHARED((npad, 16), jnp.float32),
          pltpu.VMEM_SHARED((npad, 16), jnp.float32),
      ],
      compiler_params=pltpu.CompilerParams(use_tc_tiling_on_sc=False),
  )
  def k(msg_hbm, dsti_hbm, zrows_hbm, ones_hbm, num_hbm, cnt_hbm,
        idx_v, msg_v, ones_v, acc_num, acc_cnt):
    c = lax.axis_index("c")
    s = lax.axis_index("s")
    w = s * NC + c

    pltpu.sync_copy(zrows_hbm, acc_num.at[pl.ds(s * rpt, rpt)])
    pltpu.sync_copy(zrows_hbm, acc_cnt.at[pl.ds(s * rpt, rpt)])
    pltpu.sync_copy(ones_hbm, ones_v)
    plsc.subcore_barrier()

    base0 = w * ew

    def step(j, carry):
      base = base0 + j * CHUNK
      pltpu.sync_copy(dsti_hbm.at[pl.ds(base, CHUNK)], idx_v)
      pltpu.sync_copy(msg_hbm.at[pl.ds(base, CHUNK)], msg_v)
      pltpu.sync_copy(msg_v, acc_num.at[idx_v], add=True)
      pltpu.sync_copy(ones_v, acc_cnt.at[idx_v], add=True)
      return carry

    lax.fori_loop(0, nch, step, 0)
    plsc.subcore_barrier()

    off = c * npad + s * rpt
    pltpu.sync_copy(acc_num.at[pl.ds(s * rpt, rpt)], num_hbm.at[pl.ds(off, rpt)])
    pltpu.sync_copy(acc_cnt.at[pl.ds(s * rpt, rpt)], cnt_hbm.at[pl.ds(off, rpt)])

  return k(msg, dsti, zrows, ones)


def _dense_body(a1_ref, a2_ref, w1a_ref, w1b_ref, b1_ref, w2a_ref, msg_ref):
  a1 = a1_ref[...]
  a2 = a2_ref[...]
  hpre = (jnp.dot(a1, w1a_ref[...], preferred_element_type=jnp.float32)
          + jnp.dot(a2, w1b_ref[...], preferred_element_type=jnp.float32)
          + b1_ref[...])
  h = 0.5 * hpre * (1.0 + lax.erf(hpre * 0.7071067811865476))
  x = a1[:, :16]
  g = jnp.dot(x, w2a_ref[...], preferred_element_type=jnp.float32)
  msg = g[:, 128:144]
  for kk in range(8):
    msg = msg + h[:, kk:kk + 1] * g[:, 16 * kk:16 * kk + 16]
  msg_ref[...] = msg


def _dense_call(a1, a2, w1a, w1b, b1, w2a, e_pad, blk=2048):
  grid = (e_pad // blk,)
  return pl.pallas_call(
      _dense_body,
      grid=grid,
      in_specs=[
          pl.BlockSpec((blk, 32), lambda i: (i, 0)),
          pl.BlockSpec((blk, 32), lambda i: (i, 0)),
          pl.BlockSpec((32, 8), lambda i: (0, 0)),
          pl.BlockSpec((32, 8), lambda i: (0, 0)),
          pl.BlockSpec((1, 8), lambda i: (0, 0)),
          pl.BlockSpec((16, 144), lambda i: (0, 0)),
      ],
      out_specs=pl.BlockSpec((blk, 16), lambda i: (i, 0)),
      out_shape=jax.ShapeDtypeStruct((e_pad, 16), jnp.float32),
  )(a1, a2, w1a, w1b, b1, w2a)


def _final_body(num_ref, cnt_ref, u0_ref, root_ref, out_ref):
  num = num_ref[0] + num_ref[1]
  cnt = cnt_ref[0] + cnt_ref[1]
  aggr = num / jnp.maximum(cnt, 1.0)
  out_ref[...] = aggr + jnp.dot(u0_ref[...], root_ref[...],
                                preferred_element_type=jnp.float32)


def _final_call(num_p, cnt_p, u0, root, n):
  return pl.pallas_call(
      _final_body,
      out_shape=jax.ShapeDtypeStruct((n, 16), jnp.float32),
  )(num_p, cnt_p, u0, root)


def kernel(u, grid, edge_index_one, w1, b1, w2, b2, root):
  b = u.shape[0]
  cin = u.shape[1]
  n = u.shape[2]
  cout = root.shape[1]
  dp = grid.shape[-1]
  e = edge_index_one.shape[1]

  u0 = jnp.transpose(u[0])
  npad = ((n + 1 + NS - 1) // NS + 7) // 8 * 8 * NS
  e_pad = ((e + NW * CHUNK - 1) // (NW * CHUNK)) * (NW * CHUNK)

  tbl = jnp.zeros((npad, 32), jnp.float32)
  tbl = tbl.at[:n, :cin].set(u0).at[:n, cin:cin + dp].set(grid[0])

  src = edge_index_one[0, :, 0]
  dst = edge_index_one[1, :, 0]
  pad_idx = jnp.full((e_pad - e,), n, jnp.int32)
  src_p = jnp.concatenate([src, pad_idx])
  dst_p = jnp.concatenate([dst, pad_idx])

  w1a = jnp.zeros((32, 8), jnp.float32).at[:cin + dp].set(w1[:cin + dp])
  w1b = jnp.zeros((32, 8), jnp.float32).at[:cin + dp].set(w1[cin + dp:])
  b1r = b1.reshape(1, 8)
  w2a = jnp.concatenate(
      [w2.reshape(8, cin, cout).transpose(1, 0, 2).reshape(cin, 8 * cout),
       b2.reshape(cin, cout)], axis=1)

  zrows = jnp.zeros((npad // NS, 16), jnp.float32)
  ones = jnp.ones((CHUNK, 16), jnp.float32)

  a1, a2 = _gather_call(tbl, src_p, dst_p, e_pad, npad, 32)

  msg = _dense_call(a1, a2, w1a, w1b, b1r, w2a, e_pad)

  num_f, cnt_f = _scatter_call(msg, dst_p, zrows, ones, e_pad, npad)
  num_p = num_f.reshape(NC, npad, 16)[:, :n]
  cnt_p = cnt_f.reshape(NC, npad, 16)[:, :n]

  o = _final_call(num_p, cnt_p, u0, root, n)

  out = jnp.transpose(o)[None]
  return out


# speedup vs baseline: 2.7981x; 1.1434x over previous
import functools

import jax
import jax.numpy as jnp
from jax import lax
from jax.experimental import pallas as pl
from jax.experimental.pallas import tpu as pltpu
from jax.experimental.pallas import tpu_sc as plsc

NC = 2
NS = 16
NW = NC * NS
CHUNK = 128


SUP = 5
SROWS = SUP * CHUNK


def _gather_call(tbl, src2, dst2, e_pad, npad, width):
  ew = e_pad // NW
  nch = ew // CHUNK
  nsup = ew // SROWS

  mesh = plsc.VectorSubcoreMesh(core_axis_name="c", subcore_axis_name="s")

  @functools.partial(
      pl.kernel,
      out_type=[
          jax.ShapeDtypeStruct((e_pad, width), jnp.float32),
          jax.ShapeDtypeStruct((e_pad, width), jnp.float32),
      ],
      mesh=mesh,
      scratch_types=[
          pltpu.VMEM((nch, CHUNK), jnp.int32),
          pltpu.VMEM((nch, CHUNK), jnp.int32),
          pltpu.VMEM((SROWS, width), jnp.float32),
          pltpu.VMEM((SROWS, width), jnp.float32),
          pltpu.VMEM((SROWS, width), jnp.float32),
          pltpu.VMEM((SROWS, width), jnp.float32),
          pltpu.SemaphoreType.DMA,
          pltpu.SemaphoreType.DMA,
      ],
      compiler_params=pltpu.CompilerParams(use_tc_tiling_on_sc=False),
  )
  def k(tbl_hbm, src_hbm, dst_hbm, a1_hbm, a2_hbm,
        idxs_v, idxd_v, r1a, r1b, r2a, r2b, gsem, wsem):
    w = lax.axis_index("s") * NC + lax.axis_index("c")
    base0 = w * ew
    r1 = (r1a, r1b)
    r2 = (r2a, r2b)

    pltpu.sync_copy(src_hbm.at[pl.ds(w * nch, nch)], idxs_v)
    pltpu.sync_copy(dst_hbm.at[pl.ds(w * nch, nch)], idxd_v)

    pend_w = [None, None]
    for g in range(nsup):
      b = g % 2
      if pend_w[b] is not None:
        for d in pend_w[b]:
          d.wait()
      gds = []
      for j in range(SUP):
        ch = g * SUP + j
        gds.append(pltpu.async_copy(
            tbl_hbm.at[idxs_v.at[ch]], r1[b].at[pl.ds(j * CHUNK, CHUNK)], gsem))
        gds.append(pltpu.async_copy(
            tbl_hbm.at[idxd_v.at[ch]], r2[b].at[pl.ds(j * CHUNK, CHUNK)], gsem))
      for d in gds:
        d.wait()
      base = base0 + g * SROWS
      pend_w[b] = [
          pltpu.async_copy(r1[b], a1_hbm.at[pl.ds(base, SROWS)], wsem),
          pltpu.async_copy(r2[b], a2_hbm.at[pl.ds(base, SROWS)], wsem),
      ]
    for b in (0, 1):
      if pend_w[b] is not None:
        for d in pend_w[b]:
          d.wait()

  return k(tbl, src2, dst2)


def _scatter_call(msg, dst2, zrows, ones, e_pad, npad):
  ew = e_pad // NW
  nch = ew // CHUNK
  nsup = ew // SROWS
  rpt = npad // NS

  mesh = plsc.VectorSubcoreMesh(core_axis_name="c", subcore_axis_name="s")

  @functools.partial(
      pl.kernel,
      out_type=[
          jax.ShapeDtypeStruct((NC * npad, 16), jnp.float32),
          jax.ShapeDtypeStruct((NC * npad, 16), jnp.float32),
      ],
      mesh=mesh,
      scratch_types=[
          pltpu.VMEM((nch, CHUNK), jnp.int32),
          pltpu.VMEM((SROWS, 16), jnp.float32),
          pltpu.VMEM((SROWS, 16), jnp.float32),
          pltpu.VMEM((CHUNK, 16), jnp.float32),
          pltpu.VMEM_SHARED((npad, 16), jnp.float32),
          pltpu.VMEM_SHARED((npad, 16), jnp.float32),
          pltpu.SemaphoreType.DMA,
          pltpu.SemaphoreType.DMA,
      ],
      compiler_params=pltpu.CompilerParams(use_tc_tiling_on_sc=False),
  )
  def k(msg_hbm, dst_hbm, zrows_hbm, ones_hbm, num_hbm, cnt_hbm,
        idxd_v, ma, mb, ones_v, acc_num, acc_cnt, lsem, asem):
    c = lax.axis_index("c")
    s = lax.axis_index("s")
    w = s * NC + c
    mB = (ma, mb)

    pltpu.sync_copy(zrows_hbm, acc_num.at[pl.ds(s * rpt, rpt)])
    pltpu.sync_copy(zrows_hbm, acc_cnt.at[pl.ds(s * rpt, rpt)])
    pltpu.sync_copy(ones_hbm, ones_v)
    pltpu.sync_copy(dst_hbm.at[pl.ds(w * nch, nch)], idxd_v)
    plsc.subcore_barrier()

    base0 = w * ew
    pend_add = [[], []]
    pend_load = [None, None]
    pend_load[0] = pltpu.async_copy(
        msg_hbm.at[pl.ds(base0, SROWS)], mB[0], lsem)
    for g in range(nsup):
      b = g % 2
      if g + 1 < nsup:
        nb = (g + 1) % 2
        for d in pend_add[nb]:
          d.wait()
        pend_add[nb] = []
        pend_load[nb] = pltpu.async_copy(
            msg_hbm.at[pl.ds(base0 + (g + 1) * SROWS, SROWS)], mB[nb], lsem)
      pend_load[b].wait()
      for j in range(SUP):
        ch = g * SUP + j
        pend_add[b].append(pltpu.async_copy(
            mB[b].at[pl.ds(j * CHUNK, CHUNK)], acc_num.at[idxd_v.at[ch]],
            asem, add=True))
        pend_add[b].append(pltpu.async_copy(
            ones_v, acc_cnt.at[idxd_v.at[ch]], asem, add=True))
    for b in (0, 1):
      for d in pend_add[b]:
        d.wait()
    plsc.subcore_barrier()

    off = c * npad + s * rpt
    d1 = pltpu.async_copy(acc_num.at[pl.ds(s * rpt, rpt)],
                          num_hbm.at[pl.ds(off, rpt)], lsem)
    d2 = pltpu.async_copy(acc_cnt.at[pl.ds(s * rpt, rpt)],
                          cnt_hbm.at[pl.ds(off, rpt)], lsem)
    d1.wait()
    d2.wait()

  return k(msg, dst2, zrows, ones)


def _dense_body(a1_ref, a2_ref, w1a_ref, w1b_ref, b1_ref, w2a_ref, msg_ref):
  a1 = a1_ref[...]
  a2 = a2_ref[...]
  hpre = (jnp.dot(a1, w1a_ref[...], preferred_element_type=jnp.float32)
          + jnp.dot(a2, w1b_ref[...], preferred_element_type=jnp.float32)
          + b1_ref[...])
  h = 0.5 * hpre * (1.0 + lax.erf(hpre * 0.7071067811865476))
  x = a1[:, :16]
  g = jnp.dot(x, w2a_ref[...], preferred_element_type=jnp.float32)
  msg = g[:, 128:144]
  for kk in range(8):
    msg = msg + h[:, kk:kk + 1] * g[:, 16 * kk:16 * kk + 16]
  msg_ref[...] = msg


def _dense_call(a1, a2, w1a, w1b, b1, w2a, e_pad, blk=2048):
  grid = (e_pad // blk,)
  return pl.pallas_call(
      _dense_body,
      grid=grid,
      in_specs=[
          pl.BlockSpec((blk, 32), lambda i: (i, 0)),
          pl.BlockSpec((blk, 32), lambda i: (i, 0)),
          pl.BlockSpec((32, 8), lambda i: (0, 0)),
          pl.BlockSpec((32, 8), lambda i: (0, 0)),
          pl.BlockSpec((1, 8), lambda i: (0, 0)),
          pl.BlockSpec((16, 144), lambda i: (0, 0)),
      ],
      out_specs=pl.BlockSpec((blk, 16), lambda i: (i, 0)),
      out_shape=jax.ShapeDtypeStruct((e_pad, 16), jnp.float32),
  )(a1, a2, w1a, w1b, b1, w2a)


def _final_body(num_ref, cnt_ref, u0_ref, root_ref, out_ref):
  num = num_ref[0] + num_ref[1]
  cnt = cnt_ref[0] + cnt_ref[1]
  aggr = num / jnp.maximum(cnt, 1.0)
  out_ref[...] = aggr + jnp.dot(u0_ref[...], root_ref[...],
                                preferred_element_type=jnp.float32)


def _final_call(num_p, cnt_p, u0, root, n):
  return pl.pallas_call(
      _final_body,
      out_shape=jax.ShapeDtypeStruct((n, 16), jnp.float32),
  )(num_p, cnt_p, u0, root)


def kernel(u, grid, edge_index_one, w1, b1, w2, b2, root):
  b = u.shape[0]
  cin = u.shape[1]
  n = u.shape[2]
  cout = root.shape[1]
  dp = grid.shape[-1]
  e = edge_index_one.shape[1]

  u0 = jnp.transpose(u[0])
  npad = ((n + 1 + NS - 1) // NS + 7) // 8 * 8 * NS
  e_pad = ((e + NW * CHUNK - 1) // (NW * CHUNK)) * (NW * CHUNK)

  tbl = jnp.zeros((npad, 32), jnp.float32)
  tbl = tbl.at[:n, :cin].set(u0).at[:n, cin:cin + dp].set(grid[0])

  src = edge_index_one[0, :, 0]
  dst = edge_index_one[1, :, 0]
  pad_idx = jnp.full((e_pad - e,), n, jnp.int32)
  src_p = jnp.concatenate([src, pad_idx]).reshape(e_pad // CHUNK, CHUNK)
  dst_p = jnp.concatenate([dst, pad_idx]).reshape(e_pad // CHUNK, CHUNK)

  w1a = jnp.zeros((32, 8), jnp.float32).at[:cin + dp].set(w1[:cin + dp])
  w1b = jnp.zeros((32, 8), jnp.float32).at[:cin + dp].set(w1[cin + dp:])
  b1r = b1.reshape(1, 8)
  w2a = jnp.concatenate(
      [w2.reshape(8, cin, cout).transpose(1, 0, 2).reshape(cin, 8 * cout),
       b2.reshape(cin, cout)], axis=1)

  zrows = jnp.zeros((npad // NS, 16), jnp.float32)
  ones = jnp.ones((CHUNK, 16), jnp.float32)

  a1, a2 = _gather_call(tbl, src_p, dst_p, e_pad, npad, 32)

  msg = _dense_call(a1, a2, w1a, w1b, b1r, w2a, e_pad)

  num_f, cnt_f = _scatter_call(msg, dst_p, zrows, ones, e_pad, npad)
  num_p = num_f.reshape(NC, npad, 16)[:, :n]
  cnt_p = cnt_f.reshape(NC, npad, 16)[:, :n]

  o = _final_call(num_p, cnt_p, u0, root, n)

  out = jnp.transpose(o)[None]
  return out


# speedup vs baseline: 4.0077x; 1.4323x over previous
import functools

import jax
import jax.numpy as jnp
from jax import lax
from jax.experimental import pallas as pl
from jax.experimental.pallas import tpu as pltpu
from jax.experimental.pallas import tpu_sc as plsc

NC = 2
NS = 16
NW = NC * NS
CHUNK = 128


SUP = 5
SROWS = SUP * CHUNK


def _gather_call(tbl, src2, dst2, e_pad, npad, width):
  ew = e_pad // NW
  nch = ew // CHUNK
  nsup = ew // SROWS

  mesh = plsc.VectorSubcoreMesh(core_axis_name="c", subcore_axis_name="s")

  @functools.partial(
      pl.kernel,
      out_type=[
          jax.ShapeDtypeStruct((e_pad, width), jnp.float32),
          jax.ShapeDtypeStruct((e_pad, width), jnp.float32),
      ],
      mesh=mesh,
      scratch_types=[
          pltpu.VMEM((nch, CHUNK), jnp.int32),
          pltpu.VMEM((nch, CHUNK), jnp.int32),
          pltpu.VMEM((SROWS, width), jnp.float32),
          pltpu.VMEM((SROWS, width), jnp.float32),
          pltpu.VMEM((SROWS, width), jnp.float32),
          pltpu.VMEM((SROWS, width), jnp.float32),
          pltpu.SemaphoreType.DMA,
          pltpu.SemaphoreType.DMA,
      ],
      compiler_params=pltpu.CompilerParams(use_tc_tiling_on_sc=False),
  )
  def k(tbl_hbm, src_hbm, dst_hbm, a1_hbm, a2_hbm,
        idxs_v, idxd_v, r1a, r1b, r2a, r2b, gsem, wsem):
    w = lax.axis_index("s") * NC + lax.axis_index("c")
    base0 = w * ew
    r1 = (r1a, r1b)
    r2 = (r2a, r2b)

    pltpu.sync_copy(src_hbm.at[pl.ds(w * nch, nch)], idxs_v)
    pltpu.sync_copy(dst_hbm.at[pl.ds(w * nch, nch)], idxd_v)

    pend_w = [None, None]
    for g in range(nsup):
      b = g % 2
      if pend_w[b] is not None:
        for d in pend_w[b]:
          d.wait()
      gds = []
      for j in range(SUP):
        ch = g * SUP + j
        gds.append(pltpu.async_copy(
            tbl_hbm.at[idxs_v.at[ch]], r1[b].at[pl.ds(j * CHUNK, CHUNK)], gsem))
        gds.append(pltpu.async_copy(
            tbl_hbm.at[idxd_v.at[ch]], r2[b].at[pl.ds(j * CHUNK, CHUNK)], gsem))
      for d in gds:
        d.wait()
      base = base0 + g * SROWS
      pend_w[b] = [
          pltpu.async_copy(r1[b], a1_hbm.at[pl.ds(base, SROWS)], wsem),
          pltpu.async_copy(r2[b], a2_hbm.at[pl.ds(base, SROWS)], wsem),
      ]
    for b in (0, 1):
      if pend_w[b] is not None:
        for d in pend_w[b]:
          d.wait()

  return k(tbl, src2, dst2)


def _scatter_call(msg, dst2, zrows, ones, e_pad, npad):
  ew = e_pad // NW
  nch = ew // CHUNK
  nsup = ew // SROWS
  rpt = npad // NS

  mesh = plsc.VectorSubcoreMesh(core_axis_name="c", subcore_axis_name="s")

  @functools.partial(
      pl.kernel,
      out_type=[
          jax.ShapeDtypeStruct((NC * npad, 16), jnp.float32),
          jax.ShapeDtypeStruct((NC * npad, 16), jnp.float32),
      ],
      mesh=mesh,
      scratch_types=[
          pltpu.VMEM((nch, CHUNK), jnp.int32),
          pltpu.VMEM((SROWS, 16), jnp.float32),
          pltpu.VMEM((SROWS, 16), jnp.float32),
          pltpu.VMEM((CHUNK, 16), jnp.float32),
          pltpu.VMEM_SHARED((npad, 16), jnp.float32),
          pltpu.VMEM_SHARED((npad, 16), jnp.float32),
          pltpu.SemaphoreType.DMA,
          pltpu.SemaphoreType.DMA,
      ],
      compiler_params=pltpu.CompilerParams(use_tc_tiling_on_sc=False),
  )
  def k(msg_hbm, dst_hbm, zrows_hbm, ones_hbm, num_hbm, cnt_hbm,
        idxd_v, ma, mb, ones_v, acc_num, acc_cnt, lsem, asem):
    c = lax.axis_index("c")
    s = lax.axis_index("s")
    w = s * NC + c
    mB = (ma, mb)

    pltpu.sync_copy(zrows_hbm, acc_num.at[pl.ds(s * rpt, rpt)])
    pltpu.sync_copy(zrows_hbm, acc_cnt.at[pl.ds(s * rpt, rpt)])
    pltpu.sync_copy(ones_hbm, ones_v)
    pltpu.sync_copy(dst_hbm.at[pl.ds(w * nch, nch)], idxd_v)
    plsc.subcore_barrier()

    base0 = w * ew
    pend_add = [[], []]
    pend_load = [None, None]
    pend_load[0] = pltpu.async_copy(
        msg_hbm.at[pl.ds(base0, SROWS)], mB[0], lsem)
    for g in range(nsup):
      b = g % 2
      if g + 1 < nsup:
        nb = (g + 1) % 2
        for d in pend_add[nb]:
          d.wait()
        pend_add[nb] = []
        pend_load[nb] = pltpu.async_copy(
            msg_hbm.at[pl.ds(base0 + (g + 1) * SROWS, SROWS)], mB[nb], lsem)
      pend_load[b].wait()
      for j in range(SUP):
        ch = g * SUP + j
        pend_add[b].append(pltpu.async_copy(
            mB[b].at[pl.ds(j * CHUNK, CHUNK)], acc_num.at[idxd_v.at[ch]],
            asem, add=True))
        pend_add[b].append(pltpu.async_copy(
            ones_v, acc_cnt.at[idxd_v.at[ch]], asem, add=True))
    for b in (0, 1):
      for d in pend_add[b]:
        d.wait()
    plsc.subcore_barrier()

    off = c * npad + s * rpt
    d1 = pltpu.async_copy(acc_num.at[pl.ds(s * rpt, rpt)],
                          num_hbm.at[pl.ds(off, rpt)], lsem)
    d2 = pltpu.async_copy(acc_cnt.at[pl.ds(s * rpt, rpt)],
                          cnt_hbm.at[pl.ds(off, rpt)], lsem)
    d1.wait()
    d2.wait()

  return k(msg, dst2, zrows, ones)


def _dense_body(a1_ref, a2_ref, w1a_ref, w1b_ref, b1_ref, rep8_ref, tile8_ref,
                w2f_ref, b2r_ref, msg_ref):
  a1 = a1_ref[...]
  a2 = a2_ref[...]
  hpre = (jnp.dot(a1, w1a_ref[...], preferred_element_type=jnp.float32)
          + jnp.dot(a2, w1b_ref[...], preferred_element_type=jnp.float32)
          + b1_ref[...])
  h = 0.5 * hpre * (1.0 + lax.erf(hpre * 0.7071067811865476))
  x = a1[:, :16]
  hr = jnp.dot(h, rep8_ref[...], preferred_element_type=jnp.float32)
  xt = jnp.dot(x, tile8_ref[...], preferred_element_type=jnp.float32)
  msg = (jnp.dot(hr * xt, w2f_ref[...], preferred_element_type=jnp.float32)
         + jnp.dot(x, b2r_ref[...], preferred_element_type=jnp.float32))
  msg_ref[...] = msg


def _dense_call(a1, a2, w1a, w1b, b1, rep8, tile8, w2f, b2r, e_pad, blk=2048):
  grid = (e_pad // blk,)
  return pl.pallas_call(
      _dense_body,
      grid=grid,
      in_specs=[
          pl.BlockSpec((blk, 32), lambda i: (i, 0)),
          pl.BlockSpec((blk, 32), lambda i: (i, 0)),
          pl.BlockSpec((32, 8), lambda i: (0, 0)),
          pl.BlockSpec((32, 8), lambda i: (0, 0)),
          pl.BlockSpec((1, 8), lambda i: (0, 0)),
          pl.BlockSpec((8, 128), lambda i: (0, 0)),
          pl.BlockSpec((16, 128), lambda i: (0, 0)),
          pl.BlockSpec((128, 16), lambda i: (0, 0)),
          pl.BlockSpec((16, 16), lambda i: (0, 0)),
      ],
      out_specs=pl.BlockSpec((blk, 16), lambda i: (i, 0)),
      out_shape=jax.ShapeDtypeStruct((e_pad, 16), jnp.float32),
  )(a1, a2, w1a, w1b, b1, rep8, tile8, w2f, b2r)


def _final_body(num_ref, cnt_ref, u0_ref, root_ref, out_ref):
  num = num_ref[0] + num_ref[1]
  cnt = cnt_ref[0] + cnt_ref[1]
  aggr = num / jnp.maximum(cnt, 1.0)
  out_ref[...] = aggr + jnp.dot(u0_ref[...], root_ref[...],
                                preferred_element_type=jnp.float32)


def _final_call(num_p, cnt_p, u0, root, n):
  return pl.pallas_call(
      _final_body,
      out_shape=jax.ShapeDtypeStruct((n, 16), jnp.float32),
  )(num_p, cnt_p, u0, root)


def kernel(u, grid, edge_index_one, w1, b1, w2, b2, root):
  b = u.shape[0]
  cin = u.shape[1]
  n = u.shape[2]
  cout = root.shape[1]
  dp = grid.shape[-1]
  e = edge_index_one.shape[1]

  u0 = jnp.transpose(u[0])
  npad = ((n + 1 + NS - 1) // NS + 7) // 8 * 8 * NS
  e_pad = ((e + NW * CHUNK - 1) // (NW * CHUNK)) * (NW * CHUNK)

  tbl = jnp.zeros((npad, 32), jnp.float32)
  tbl = tbl.at[:n, :cin].set(u0).at[:n, cin:cin + dp].set(grid[0])

  src = edge_index_one[0, :, 0]
  dst = edge_index_one[1, :, 0]
  pad_idx = jnp.full((e_pad - e,), n, jnp.int32)
  src_p = jnp.concatenate([src, pad_idx]).reshape(e_pad // CHUNK, CHUNK)
  dst_p = jnp.concatenate([dst, pad_idx]).reshape(e_pad // CHUNK, CHUNK)

  w1a = jnp.zeros((32, 8), jnp.float32).at[:cin + dp].set(w1[:cin + dp])
  w1b = jnp.zeros((32, 8), jnp.float32).at[:cin + dp].set(w1[cin + dp:])
  b1r = b1.reshape(1, 8)
  rep8 = jnp.kron(jnp.eye(8, dtype=jnp.float32),
                  jnp.ones((1, cout), jnp.float32))
  tile8 = jnp.tile(jnp.eye(cin, dtype=jnp.float32), (1, 8))
  w2f = w2.reshape(8 * cin, cout)
  b2r = b2.reshape(cin, cout)

  zrows = jnp.zeros((npad // NS, 16), jnp.float32)
  ones = jnp.ones((CHUNK, 16), jnp.float32)

  a1, a2 = _gather_call(tbl, src_p, dst_p, e_pad, npad, 32)

  msg = _dense_call(a1, a2, w1a, w1b, b1r, rep8, tile8, w2f, b2r, e_pad)

  num_f, cnt_f = _scatter_call(msg, dst_p, zrows, ones, e_pad, npad)
  num_p = num_f.reshape(NC, npad, 16)[:, :n]
  cnt_p = cnt_f.reshape(NC, npad, 16)[:, :n]

  o = _final_call(num_p, cnt_p, u0, root, n)

  out = jnp.transpose(o)[None]
  return out


# speedup vs baseline: 4.7042x; 1.1738x over previous
import functools

import jax
import jax.numpy as jnp
from jax import lax
from jax.experimental import pallas as pl
from jax.experimental.pallas import tpu as pltpu
from jax.experimental.pallas import tpu_sc as plsc

NC = 2
NS = 16
NW = NC * NS
CHUNK = 128


SUP = 5
SROWS = SUP * CHUNK


def _gather_call(tbl, src2, dst2, e_pad, npad, width):
  ew = e_pad // NW
  nch = ew // CHUNK
  nsup = ew // SROWS

  mesh = plsc.VectorSubcoreMesh(core_axis_name="c", subcore_axis_name="s")

  @functools.partial(
      pl.kernel,
      out_type=[
          jax.ShapeDtypeStruct((e_pad, width), jnp.float32),
          jax.ShapeDtypeStruct((e_pad, width), jnp.float32),
      ],
      mesh=mesh,
      scratch_types=[
          pltpu.VMEM((nch, CHUNK), jnp.int32),
          pltpu.VMEM((nch, CHUNK), jnp.int32),
          pltpu.VMEM((SROWS, width), jnp.float32),
          pltpu.VMEM((SROWS, width), jnp.float32),
          pltpu.VMEM((SROWS, width), jnp.float32),
          pltpu.VMEM((SROWS, width), jnp.float32),
          pltpu.VMEM_SHARED((npad, width), jnp.float32),
          pltpu.SemaphoreType.DMA,
          pltpu.SemaphoreType.DMA,
      ],
      compiler_params=pltpu.CompilerParams(use_tc_tiling_on_sc=False),
  )
  def k(tbl_hbm, src_hbm, dst_hbm, a1_hbm, a2_hbm,
        idxs_v, idxd_v, r1a, r1b, r2a, r2b, tbl_sp, gsem, wsem):
    s = lax.axis_index("s")
    w = s * NC + lax.axis_index("c")
    base0 = w * ew
    r1 = (r1a, r1b)
    r2 = (r2a, r2b)

    trows = npad // NS
    pltpu.sync_copy(tbl_hbm.at[pl.ds(s * trows, trows)],
                    tbl_sp.at[pl.ds(s * trows, trows)])
    pltpu.sync_copy(src_hbm.at[pl.ds(w * nch, nch)], idxs_v)
    pltpu.sync_copy(dst_hbm.at[pl.ds(w * nch, nch)], idxd_v)
    plsc.subcore_barrier()

    pend_w = [None, None]
    for g in range(nsup):
      b = g % 2
      if pend_w[b] is not None:
        for d in pend_w[b]:
          d.wait()
      gds = []
      for j in range(SUP):
        ch = g * SUP + j
        gds.append(pltpu.async_copy(
            tbl_sp.at[idxs_v.at[ch]], r1[b].at[pl.ds(j * CHUNK, CHUNK)], gsem))
        gds.append(pltpu.async_copy(
            tbl_sp.at[idxd_v.at[ch]], r2[b].at[pl.ds(j * CHUNK, CHUNK)], gsem))
      for d in gds:
        d.wait()
      base = base0 + g * SROWS
      pend_w[b] = [
          pltpu.async_copy(r1[b], a1_hbm.at[pl.ds(base, SROWS)], wsem),
          pltpu.async_copy(r2[b], a2_hbm.at[pl.ds(base, SROWS)], wsem),
      ]
    for b in (0, 1):
      if pend_w[b] is not None:
        for d in pend_w[b]:
          d.wait()

  return k(tbl, src2, dst2)


def _scatter_call(msg, dst2, zrows, ones, e_pad, npad):
  ew = e_pad // NW
  nch = ew // CHUNK
  nsup = ew // SROWS
  rpt = npad // NS

  mesh = plsc.VectorSubcoreMesh(core_axis_name="c", subcore_axis_name="s")

  @functools.partial(
      pl.kernel,
      out_type=[
          jax.ShapeDtypeStruct((NC * npad, 16), jnp.float32),
          jax.ShapeDtypeStruct((NC * npad, 16), jnp.float32),
      ],
      mesh=mesh,
      scratch_types=[
          pltpu.VMEM((nch, CHUNK), jnp.int32),
          pltpu.VMEM((SROWS, 16), jnp.float32),
          pltpu.VMEM((SROWS, 16), jnp.float32),
          pltpu.VMEM((CHUNK, 16), jnp.float32),
          pltpu.VMEM_SHARED((npad, 16), jnp.float32),
          pltpu.VMEM_SHARED((npad, 16), jnp.float32),
          pltpu.SemaphoreType.DMA,
          pltpu.SemaphoreType.DMA,
      ],
      compiler_params=pltpu.CompilerParams(use_tc_tiling_on_sc=False),
  )
  def k(msg_hbm, dst_hbm, zrows_hbm, ones_hbm, num_hbm, cnt_hbm,
        idxd_v, ma, mb, ones_v, acc_num, acc_cnt, lsem, asem):
    c = lax.axis_index("c")
    s = lax.axis_index("s")
    w = s * NC + c
    mB = (ma, mb)

    pltpu.sync_copy(zrows_hbm, acc_num.at[pl.ds(s * rpt, rpt)])
    pltpu.sync_copy(zrows_hbm, acc_cnt.at[pl.ds(s * rpt, rpt)])
    pltpu.sync_copy(ones_hbm, ones_v)
    pltpu.sync_copy(dst_hbm.at[pl.ds(w * nch, nch)], idxd_v)
    plsc.subcore_barrier()

    base0 = w * ew
    pend_add = [[], []]
    pend_load = [None, None]
    pend_load[0] = pltpu.async_copy(
        msg_hbm.at[pl.ds(base0, SROWS)], mB[0], lsem)
    for g in range(nsup):
      b = g % 2
      if g + 1 < nsup:
        nb = (g + 1) % 2
        for d in pend_add[nb]:
          d.wait()
        pend_add[nb] = []
        pend_load[nb] = pltpu.async_copy(
            msg_hbm.at[pl.ds(base0 + (g + 1) * SROWS, SROWS)], mB[nb], lsem)
      pend_load[b].wait()
      for j in range(SUP):
        ch = g * SUP + j
        pend_add[b].append(pltpu.async_copy(
            mB[b].at[pl.ds(j * CHUNK, CHUNK)], acc_num.at[idxd_v.at[ch]],
            asem, add=True))
        pend_add[b].append(pltpu.async_copy(
            ones_v, acc_cnt.at[idxd_v.at[ch]], asem, add=True))
    for b in (0, 1):
      for d in pend_add[b]:
        d.wait()
    plsc.subcore_barrier()

    off = c * npad + s * rpt
    d1 = pltpu.async_copy(acc_num.at[pl.ds(s * rpt, rpt)],
                          num_hbm.at[pl.ds(off, rpt)], lsem)
    d2 = pltpu.async_copy(acc_cnt.at[pl.ds(s * rpt, rpt)],
                          cnt_hbm.at[pl.ds(off, rpt)], lsem)
    d1.wait()
    d2.wait()

  return k(msg, dst2, zrows, ones)


def _dense_body(a1_ref, a2_ref, w1a_ref, w1b_ref, b1_ref, rep8_ref, tile8_ref,
                w2f_ref, b2r_ref, msg_ref):
  a1 = a1_ref[...]
  a2 = a2_ref[...]
  hpre = (jnp.dot(a1, w1a_ref[...], preferred_element_type=jnp.float32)
          + jnp.dot(a2, w1b_ref[...], preferred_element_type=jnp.float32)
          + b1_ref[...])
  h = 0.5 * hpre * (1.0 + lax.erf(hpre * 0.7071067811865476))
  x = a1[:, :16]
  hr = jnp.dot(h, rep8_ref[...], preferred_element_type=jnp.float32)
  xt = jnp.dot(x, tile8_ref[...], preferred_element_type=jnp.float32)
  msg = (jnp.dot(hr * xt, w2f_ref[...], preferred_element_type=jnp.float32)
         + jnp.dot(x, b2r_ref[...], preferred_element_type=jnp.float32))
  msg_ref[...] = msg


def _dense_call(a1, a2, w1a, w1b, b1, rep8, tile8, w2f, b2r, e_pad, blk=2048):
  grid = (e_pad // blk,)
  return pl.pallas_call(
      _dense_body,
      grid=grid,
      in_specs=[
          pl.BlockSpec((blk, 32), lambda i: (i, 0)),
          pl.BlockSpec((blk, 32), lambda i: (i, 0)),
          pl.BlockSpec((32, 8), lambda i: (0, 0)),
          pl.BlockSpec((32, 8), lambda i: (0, 0)),
          pl.BlockSpec((1, 8), lambda i: (0, 0)),
          pl.BlockSpec((8, 128), lambda i: (0, 0)),
          pl.BlockSpec((16, 128), lambda i: (0, 0)),
          pl.BlockSpec((128, 16), lambda i: (0, 0)),
          pl.BlockSpec((16, 16), lambda i: (0, 0)),
      ],
      out_specs=pl.BlockSpec((blk, 16), lambda i: (i, 0)),
      out_shape=jax.ShapeDtypeStruct((e_pad, 16), jnp.float32),
  )(a1, a2, w1a, w1b, b1, rep8, tile8, w2f, b2r)


def _final_body(num_ref, cnt_ref, u0_ref, root_ref, out_ref):
  num = num_ref[0] + num_ref[1]
  cnt = cnt_ref[0] + cnt_ref[1]
  aggr = num / jnp.maximum(cnt, 1.0)
  out_ref[...] = aggr + jnp.dot(u0_ref[...], root_ref[...],
                                preferred_element_type=jnp.float32)


def _final_call(num_p, cnt_p, u0, root, n):
  return pl.pallas_call(
      _final_body,
      out_shape=jax.ShapeDtypeStruct((n, 16), jnp.float32),
  )(num_p, cnt_p, u0, root)


def kernel(u, grid, edge_index_one, w1, b1, w2, b2, root):
  b = u.shape[0]
  cin = u.shape[1]
  n = u.shape[2]
  cout = root.shape[1]
  dp = grid.shape[-1]
  e = edge_index_one.shape[1]

  u0 = jnp.transpose(u[0])
  npad = ((n + 1 + NS - 1) // NS + 7) // 8 * 8 * NS
  e_pad = ((e + NW * CHUNK - 1) // (NW * CHUNK)) * (NW * CHUNK)

  tbl = jnp.zeros((npad, 32), jnp.float32)
  tbl = tbl.at[:n, :cin].set(u0).at[:n, cin:cin + dp].set(grid[0])

  src = edge_index_one[0, :, 0]
  dst = edge_index_one[1, :, 0]
  pad_idx = jnp.full((e_pad - e,), n, jnp.int32)
  src_p = jnp.concatenate([src, pad_idx]).reshape(e_pad // CHUNK, CHUNK)
  dst_p = jnp.concatenate([dst, pad_idx]).reshape(e_pad // CHUNK, CHUNK)

  w1a = jnp.zeros((32, 8), jnp.float32).at[:cin + dp].set(w1[:cin + dp])
  w1b = jnp.zeros((32, 8), jnp.float32).at[:cin + dp].set(w1[cin + dp:])
  b1r = b1.reshape(1, 8)
  rep8 = jnp.kron(jnp.eye(8, dtype=jnp.float32),
                  jnp.ones((1, cout), jnp.float32))
  tile8 = jnp.tile(jnp.eye(cin, dtype=jnp.float32), (1, 8))
  w2f = w2.reshape(8 * cin, cout)
  b2r = b2.reshape(cin, cout)

  zrows = jnp.zeros((npad // NS, 16), jnp.float32)
  ones = jnp.ones((CHUNK, 16), jnp.float32)

  a1, a2 = _gather_call(tbl, src_p, dst_p, e_pad, npad, 32)

  msg = _dense_call(a1, a2, w1a, w1b, b1r, rep8, tile8, w2f, b2r, e_pad)

  num_f, cnt_f = _scatter_call(msg, dst_p, zrows, ones, e_pad, npad)
  num_p = num_f.reshape(NC, npad, 16)[:, :n]
  cnt_p = cnt_f.reshape(NC, npad, 16)[:, :n]

  o = _final_call(num_p, cnt_p, u0, root, n)

  out = jnp.transpose(o)[None]
  return out


# speedup vs baseline: 5.1764x; 1.1004x over previous
import functools

import jax
import jax.numpy as jnp
from jax import lax
from jax.experimental import pallas as pl
from jax.experimental.pallas import tpu as pltpu
from jax.experimental.pallas import tpu_sc as plsc

NC = 2
NS = 16
NW = NC * NS
CHUNK = 128


SUP = 5
SROWS = SUP * CHUNK


def _gather_call(tbl, src2, dst2, e_pad, npad, width):
  ew = e_pad // NW
  nch = ew // CHUNK
  nsup = ew // SROWS

  mesh = plsc.VectorSubcoreMesh(core_axis_name="c", subcore_axis_name="s")

  @functools.partial(
      pl.kernel,
      out_type=[
          jax.ShapeDtypeStruct((e_pad, width), jnp.float32),
          jax.ShapeDtypeStruct((e_pad, width), jnp.float32),
      ],
      mesh=mesh,
      scratch_types=[
          pltpu.VMEM((nch, CHUNK), jnp.int32),
          pltpu.VMEM((nch, CHUNK), jnp.int32),
          pltpu.VMEM((SROWS, width), jnp.float32),
          pltpu.VMEM((SROWS, width), jnp.float32),
          pltpu.VMEM((SROWS, width), jnp.float32),
          pltpu.VMEM((SROWS, width), jnp.float32),
          pltpu.VMEM_SHARED((npad, width), jnp.float32),
          pltpu.SemaphoreType.DMA,
          pltpu.SemaphoreType.DMA,
      ],
      compiler_params=pltpu.CompilerParams(use_tc_tiling_on_sc=False),
  )
  def k(tbl_hbm, src_hbm, dst_hbm, a1_hbm, a2_hbm,
        idxs_v, idxd_v, r1a, r1b, r2a, r2b, tbl_sp, gsem, wsem):
    s = lax.axis_index("s")
    w = s * NC + lax.axis_index("c")
    base0 = w * ew
    r1 = (r1a, r1b)
    r2 = (r2a, r2b)

    trows = npad // NS
    pltpu.sync_copy(tbl_hbm.at[pl.ds(s * trows, trows)],
                    tbl_sp.at[pl.ds(s * trows, trows)])
    pltpu.sync_copy(src_hbm.at[pl.ds(w * nch, nch)], idxs_v)
    pltpu.sync_copy(dst_hbm.at[pl.ds(w * nch, nch)], idxd_v)
    plsc.subcore_barrier()

    pend_w = [None, None]
    for g in range(nsup):
      b = g % 2
      if pend_w[b] is not None:
        for d in pend_w[b]:
          d.wait()
      gds = []
      for j in range(SUP):
        ch = g * SUP + j
        gds.append(pltpu.async_copy(
            tbl_sp.at[idxs_v.at[ch]], r1[b].at[pl.ds(j * CHUNK, CHUNK)], gsem))
        gds.append(pltpu.async_copy(
            tbl_sp.at[idxd_v.at[ch]], r2[b].at[pl.ds(j * CHUNK, CHUNK)], gsem))
      for d in gds:
        d.wait()
      base = base0 + g * SROWS
      pend_w[b] = [
          pltpu.async_copy(r1[b], a1_hbm.at[pl.ds(base, SROWS)], wsem),
          pltpu.async_copy(r2[b], a2_hbm.at[pl.ds(base, SROWS)], wsem),
      ]
    for b in (0, 1):
      if pend_w[b] is not None:
        for d in pend_w[b]:
          d.wait()

  return k(tbl, src2, dst2)


def _scatter_call(msg, dst2, zrows, ones, e_pad, npad):
  ew = e_pad // NW
  nch = ew // CHUNK
  nsup = ew // SROWS
  rpt = npad // NS

  mesh = plsc.VectorSubcoreMesh(core_axis_name="c", subcore_axis_name="s")

  @functools.partial(
      pl.kernel,
      out_type=[
          jax.ShapeDtypeStruct((NC * npad, 16), jnp.float32),
          jax.ShapeDtypeStruct((NC * npad, 16), jnp.float32),
      ],
      mesh=mesh,
      scratch_types=[
          pltpu.VMEM((nch, CHUNK), jnp.int32),
          pltpu.VMEM((SROWS, 16), jnp.float32),
          pltpu.VMEM((SROWS, 16), jnp.float32),
          pltpu.VMEM((CHUNK, 16), jnp.float32),
          pltpu.VMEM_SHARED((npad, 16), jnp.float32),
          pltpu.VMEM_SHARED((npad, 16), jnp.float32),
          pltpu.SemaphoreType.DMA,
          pltpu.SemaphoreType.DMA,
      ],
      compiler_params=pltpu.CompilerParams(use_tc_tiling_on_sc=False),
  )
  def k(msg_hbm, dst_hbm, zrows_hbm, ones_hbm, num_hbm, cnt_hbm,
        idxd_v, ma, mb, ones_v, acc_num, acc_cnt, lsem, asem):
    c = lax.axis_index("c")
    s = lax.axis_index("s")
    w = s * NC + c
    mB = (ma, mb)

    pltpu.sync_copy(zrows_hbm, acc_num.at[pl.ds(s * rpt, rpt)])
    pltpu.sync_copy(zrows_hbm, acc_cnt.at[pl.ds(s * rpt, rpt)])
    pltpu.sync_copy(ones_hbm, ones_v)
    pltpu.sync_copy(dst_hbm.at[pl.ds(w * nch, nch)], idxd_v)
    plsc.subcore_barrier()

    base0 = w * ew
    pend_add = [[], []]
    pend_load = [None, None]
    pend_load[0] = pltpu.async_copy(
        msg_hbm.at[pl.ds(base0, SROWS)], mB[0], lsem)
    for g in range(nsup):
      b = g % 2
      if g + 1 < nsup:
        nb = (g + 1) % 2
        for d in pend_add[nb]:
          d.wait()
        pend_add[nb] = []
        pend_load[nb] = pltpu.async_copy(
            msg_hbm.at[pl.ds(base0 + (g + 1) * SROWS, SROWS)], mB[nb], lsem)
      pend_load[b].wait()
      for j in range(SUP):
        ch = g * SUP + j
        pend_add[b].append(pltpu.async_copy(
            mB[b].at[pl.ds(j * CHUNK, CHUNK)], acc_num.at[idxd_v.at[ch]],
            asem, add=True))
        pend_add[b].append(pltpu.async_copy(
            ones_v, acc_cnt.at[idxd_v.at[ch]], asem, add=True))
    for b in (0, 1):
      for d in pend_add[b]:
        d.wait()
    plsc.subcore_barrier()

    off = c * npad + s * rpt
    d1 = pltpu.async_copy(acc_num.at[pl.ds(s * rpt, rpt)],
                          num_hbm.at[pl.ds(off, rpt)], lsem)
    d2 = pltpu.async_copy(acc_cnt.at[pl.ds(s * rpt, rpt)],
                          cnt_hbm.at[pl.ds(off, rpt)], lsem)
    d1.wait()
    d2.wait()

  return k(msg, dst2, zrows, ones)


def _dense_body(a1_ref, a2_ref, w1a_ref, w1b_ref, b1_ref, rep8_ref, tile8_ref,
                w2f_ref, b2r_ref, msg_ref):
  a1 = a1_ref[...]
  a2 = a2_ref[...]
  hpre = (jnp.dot(a1, w1a_ref[...], preferred_element_type=jnp.float32)
          + jnp.dot(a2, w1b_ref[...], preferred_element_type=jnp.float32)
          + b1_ref[...])
  h = 0.5 * hpre * (1.0 + lax.erf(hpre * 0.7071067811865476))
  x = a1[:, :16]
  hr = jnp.dot(h, rep8_ref[...], preferred_element_type=jnp.float32)
  xt = jnp.dot(x, tile8_ref[...], preferred_element_type=jnp.float32)
  msg = (jnp.dot(hr * xt, w2f_ref[...], preferred_element_type=jnp.float32)
         + jnp.dot(x, b2r_ref[...], preferred_element_type=jnp.float32))
  msg_ref[...] = msg


def _dense_call(a1, a2, w1a, w1b, b1, rep8, tile8, w2f, b2r, e_pad, blk=4096):
  grid = (e_pad // blk,)
  return pl.pallas_call(
      _dense_body,
      grid=grid,
      in_specs=[
          pl.BlockSpec((blk, 32), lambda i: (i, 0)),
          pl.BlockSpec((blk, 32), lambda i: (i, 0)),
          pl.BlockSpec((32, 8), lambda i: (0, 0)),
          pl.BlockSpec((32, 8), lambda i: (0, 0)),
          pl.BlockSpec((1, 8), lambda i: (0, 0)),
          pl.BlockSpec((8, 128), lambda i: (0, 0)),
          pl.BlockSpec((16, 128), lambda i: (0, 0)),
          pl.BlockSpec((128, 16), lambda i: (0, 0)),
          pl.BlockSpec((16, 16), lambda i: (0, 0)),
      ],
      out_specs=pl.BlockSpec((blk, 16), lambda i: (i, 0)),
      out_shape=jax.ShapeDtypeStruct((e_pad, 16), jnp.float32),
  )(a1, a2, w1a, w1b, b1, rep8, tile8, w2f, b2r)


def _final_call(num_f, cnt_f, u0, root, n, npad):
  def body(num_ref, cnt_ref, u0_ref, root_ref, out_ref):
    num = num_ref[0:n] + num_ref[npad:npad + n]
    cnt = cnt_ref[0:n] + cnt_ref[npad:npad + n]
    aggr = num / jnp.maximum(cnt, 1.0)
    o = aggr + jnp.dot(u0_ref[...], root_ref[...],
                       preferred_element_type=jnp.float32)
    out_ref[...] = jnp.transpose(o)

  return pl.pallas_call(
      body,
      out_shape=jax.ShapeDtypeStruct((16, n), jnp.float32),
  )(num_f, cnt_f, u0, root)


def kernel(u, grid, edge_index_one, w1, b1, w2, b2, root):
  b = u.shape[0]
  cin = u.shape[1]
  n = u.shape[2]
  cout = root.shape[1]
  dp = grid.shape[-1]
  e = edge_index_one.shape[1]

  u0 = jnp.transpose(u[0])
  npad = ((n + 1 + NS - 1) // NS + 7) // 8 * 8 * NS
  e_pad = ((e + NW * CHUNK - 1) // (NW * CHUNK)) * (NW * CHUNK)

  tbl = jnp.zeros((npad, 32), jnp.float32)
  tbl = tbl.at[:n, :cin].set(u0).at[:n, cin:cin + dp].set(grid[0])

  src = edge_index_one[0, :, 0]
  dst = edge_index_one[1, :, 0]
  pad_idx = jnp.full((e_pad - e,), n, jnp.int32)
  src_p = jnp.concatenate([src, pad_idx]).reshape(e_pad // CHUNK, CHUNK)
  dst_p = jnp.concatenate([dst, pad_idx]).reshape(e_pad // CHUNK, CHUNK)

  w1a = jnp.zeros((32, 8), jnp.float32).at[:cin + dp].set(w1[:cin + dp])
  w1b = jnp.zeros((32, 8), jnp.float32).at[:cin + dp].set(w1[cin + dp:])
  b1r = b1.reshape(1, 8)
  rep8 = jnp.kron(jnp.eye(8, dtype=jnp.float32),
                  jnp.ones((1, cout), jnp.float32))
  tile8 = jnp.tile(jnp.eye(cin, dtype=jnp.float32), (1, 8))
  w2f = w2.reshape(8 * cin, cout)
  b2r = b2.reshape(cin, cout)

  zrows = jnp.zeros((npad // NS, 16), jnp.float32)
  ones = jnp.ones((CHUNK, 16), jnp.float32)

  a1, a2 = _gather_call(tbl, src_p, dst_p, e_pad, npad, 32)

  msg = _dense_call(a1, a2, w1a, w1b, b1r, rep8, tile8, w2f, b2r, e_pad)

  num_f, cnt_f = _scatter_call(msg, dst_p, zrows, ones, e_pad, npad)

  o = _final_call(num_f, cnt_f, u0, root, n, npad)

  return o[None]
